# baseline (speedup 1.0000x reference)
import functools

import jax
import jax.numpy as jnp
from jax import lax
from jax.experimental import pallas as pl
from jax.experimental.pallas import tpu as pltpu
from jax.experimental.pallas import tpu_sc as plsc

B, S, DIM = 1, 2048, 768
FFN = int(DIM * 2.0)
E, K = 8, 2
T = B * S
BT = 256
BLK = 256

H = 2
TH = T // H
NIH = TH // BT
NTH = (TH * K + E * (BLK - 1) + BLK - 1) // BLK
PH = NTH * BLK

NC, NS, L = 2, 16, 16
NW = NC * NS
TWH = TH // NW


def _meta_kernel(x_ref, wr_ref, d0_ref, d1_ref, w0_ref, w1_ref, tile_e_ref,
                 carry_ref, meta_ref, base_ref):
    ph = pl.program_id(0)
    i = pl.program_id(1)
    cols = pl.ds(i * BT, BT)
    srow = lax.broadcasted_iota(jnp.int32, (E, BT), 0)

    @pl.when(ph == 0)
    def _phase0():
        x_t = x_ref[...]
        logits_tm = jnp.dot(x_t, wr_ref[...],
                            preferred_element_type=jnp.float32)
        logits = jnp.transpose(logits_tm)
        m = jnp.max(logits, axis=0, keepdims=True)
        ex = jnp.exp(logits - m)
        probs = ex / jnp.sum(ex, axis=0, keepdims=True)
        v1 = jnp.max(probs, axis=0, keepdims=True)
        i1 = jnp.min(jnp.where(probs == v1, srow, E), axis=0, keepdims=True)
        mask1 = srow == i1
        probs2 = jnp.where(mask1, -jnp.inf, probs)
        v2 = jnp.max(probs2, axis=0, keepdims=True)
        i2 = jnp.min(jnp.where(probs2 == v2, srow, E), axis=0, keepdims=True)
        mask2 = srow == i2
        onehot = (mask1 | mask2).astype(jnp.float32)

        @pl.when(i == 0)
        def _init():
            carry_ref[...] = jnp.zeros_like(carry_ref)

        ri = lax.broadcasted_iota(jnp.int32, (BT, BT), 0)
        cj = lax.broadcasted_iota(jnp.int32, (BT, BT), 1)
        utri = (ri < cj).astype(jnp.float32)
        cex = jnp.dot(onehot, utri, preferred_element_type=jnp.float32)
        cex = cex + carry_ref[...]
        carry_ref[...] += jnp.sum(onehot, axis=1, keepdims=True)

        r0 = jnp.sum(jnp.where(mask1, cex, 0.0), axis=0, keepdims=True)
        r1 = jnp.sum(jnp.where(mask2, cex, 0.0), axis=0, keepdims=True)
        meta_ref[:, cols] = jnp.concatenate(
            [r0, r1, i1.astype(jnp.float32), i2.astype(jnp.float32), v1, v2,
             jnp.zeros((2, BT), jnp.float32)], axis=0)

    @pl.when(ph == 1)
    def _phase1():
        @pl.when(i == 0)
        def _bases():
            c = carry_ref[...]
            pc = jnp.floor((c + (BLK - 1)) / BLK) * BLK
            eA = lax.broadcasted_iota(jnp.int32, (E, E), 0)
            eB = lax.broadcasted_iota(jnp.int32, (E, E), 1)
            ltri = (eB < eA).astype(jnp.float32)
            base_ref[...] = jnp.dot(ltri, pc,
                                    preferred_element_type=jnp.float32)
            total = jnp.sum(pc)
            mm = lax.broadcasted_iota(jnp.int32, (E, NTH + 1), 1) * BLK
            mmc = jnp.minimum(mm.astype(jnp.float32), total - BLK)
            cmp = (jnp.broadcast_to(base_ref[...], (E, NTH + 1)) <= mmc
                   ).astype(jnp.float32)
            te_raw = jnp.sum(cmp, axis=0, keepdims=True) - 1.0
            mcol = lax.broadcasted_iota(jnp.int32, (1, NTH + 1), 1)
            te = jnp.where(mcol == NTH, total * (1.0 / BLK), te_raw)
            tile_e_ref[...] = te.astype(jnp.int32)

        slab = meta_ref[:, cols]

        def getr(c):
            return jnp.sum(jnp.where(srow == c, slab, 0.0), axis=0,
                           keepdims=True)

        r0, r1 = getr(0), getr(1)
        i1, i2 = getr(2).astype(jnp.int32), getr(3).astype(jnp.int32)
        v1, v2 = getr(4), getr(5)
        baseb = jnp.broadcast_to(base_ref[...], (E, BT))
        b0 = jnp.sum(jnp.where(srow == i1, baseb, 0.0), axis=0, keepdims=True)
        b1 = jnp.sum(jnp.where(srow == i2, baseb, 0.0), axis=0, keepdims=True)
        d0_ref[...] = (b0 + r0).astype(jnp.int32).reshape(1, 1, BT)
        d1_ref[...] = (b1 + r1).astype(jnp.int32).reshape(1, 1, BT)
        mrows = jnp.concatenate([v1, v2], axis=0)
        tcol = jnp.transpose(mrows)
        col2 = lax.broadcasted_iota(jnp.int32, tcol.shape, 1)

        def getcol(c):
            return jnp.sum(jnp.where(col2 == c, tcol, 0.0), axis=1,
                           keepdims=True)

        w0_ref[...] = jnp.broadcast_to(getcol(0), (BT, L))
        w1_ref[...] = jnp.broadcast_to(getcol(1), (BT, L))


def _run_meta(xf, Wr, h):
    return pl.pallas_call(
        _meta_kernel,
        grid=(2, NIH),
        in_specs=[
            pl.BlockSpec((BT, DIM), lambda p, i: (h * NIH + i, 0)),
            pl.BlockSpec((DIM, E), lambda p, i: (0, 0)),
        ],
        out_specs=[
            pl.BlockSpec((1, 1, BT), lambda p, i: (i, 0, 0)),
            pl.BlockSpec((1, 1, BT), lambda p, i: (i, 0, 0)),
            pl.BlockSpec((BT, L), lambda p, i: (i, 0)),
            pl.BlockSpec((BT, L), lambda p, i: (i, 0)),
            pl.BlockSpec((1, NTH + 1), lambda p, i: (0, 0)),
        ],
        out_shape=[
            jax.ShapeDtypeStruct((NIH, 1, BT), jnp.int32),
            jax.ShapeDtypeStruct((NIH, 1, BT), jnp.int32),
            jax.ShapeDtypeStruct((TH, L), jnp.float32),
            jax.ShapeDtypeStruct((TH, L), jnp.float32),
            jax.ShapeDtypeStruct((1, NTH + 1), jnp.int32),
        ],
        scratch_shapes=[
            pltpu.VMEM((E, 1), jnp.float32),
            pltpu.VMEM((E, TH), jnp.float32),
            pltpu.VMEM((E, 1), jnp.float32),
        ],
        compiler_params=pltpu.CompilerParams(
            dimension_semantics=("arbitrary", "arbitrary"),
        ),
    )(xf, Wr)


def _gemm_kernel(te_ref, xs_ref, w1_ref, b1_ref, w2_ref, b2_ref,
                 w3_ref, b3_ref, out_ref):
    m = pl.program_id(0)

    @pl.when(m < te_ref[NTH])
    def _compute():
        x_t = xs_ref[...]
        h1 = jnp.dot(x_t, w1_ref[0],
                     preferred_element_type=jnp.float32) + b1_ref[0]
        h2 = jnp.dot(x_t, w2_ref[0],
                     preferred_element_type=jnp.float32) + b2_ref[0]
        h = h1 * (1.0 / (1.0 + jnp.exp(-h1))) * h2
        y = jnp.dot(h, w3_ref[0],
                    preferred_element_type=jnp.float32) + b3_ref[0]
        out_ref[...] = y


def _run_gemm(tile_e, xs, W1, b1, W2, b2, W3, b3):
    grid_spec = pltpu.PrefetchScalarGridSpec(
        num_scalar_prefetch=1,
        grid=(NTH,),
        in_specs=[
            pl.BlockSpec((BLK, DIM), lambda m, te: (m, 0)),
            pl.BlockSpec((1, DIM, FFN), lambda m, te: (te[m], 0, 0)),
            pl.BlockSpec((1, 1, FFN), lambda m, te: (te[m], 0, 0)),
            pl.BlockSpec((1, DIM, FFN), lambda m, te: (te[m], 0, 0)),
            pl.BlockSpec((1, 1, FFN), lambda m, te: (te[m], 0, 0)),
            pl.BlockSpec((1, FFN, DIM), lambda m, te: (te[m], 0, 0)),
            pl.BlockSpec((1, 1, DIM), lambda m, te: (te[m], 0, 0)),
        ],
        out_specs=pl.BlockSpec((BLK, DIM), lambda m, te: (m, 0)),
    )
    return pl.pallas_call(
        _gemm_kernel,
        grid_spec=grid_spec,
        out_shape=jax.ShapeDtypeStruct((PH, DIM), jnp.float32),
        compiler_params=pltpu.CompilerParams(
            dimension_semantics=("arbitrary",),
        ),
    )(tile_e, xs, W1, b1[:, None, :], W2, b2[:, None, :], W3, b3[:, None, :])


def _make_dispatch(h):
    mesh = plsc.VectorSubcoreMesh(core_axis_name="c", subcore_axis_name="s")

    @functools.partial(
        pl.kernel, mesh=mesh,
        out_type=jax.ShapeDtypeStruct((PH, DIM), jnp.float32),
        scratch_types=[
            pltpu.VMEM((TWH,), jnp.int32),
            pltpu.VMEM((TWH,), jnp.int32),
            pltpu.VMEM((TWH, DIM), jnp.float32),
            pltpu.SemaphoreType.DMA,
        ],
    )
    def dispatch(d0_hbm, d1_hbm, x_hbm, xs_hbm, d0_v, d1_v, rows_v, sem):
        wid = lax.axis_index("s") * NC + lax.axis_index("c")
        base = wid * TWH
        pltpu.sync_copy(d0_hbm.at[pl.ds(base, TWH)], d0_v)
        pltpu.sync_copy(d1_hbm.at[pl.ds(base, TWH)], d1_v)
        pltpu.sync_copy(x_hbm.at[pl.ds(h * TH + base, TWH)], rows_v)
        c0 = pltpu.async_copy(rows_v, xs_hbm.at[d0_v], sem)
        c1 = pltpu.async_copy(rows_v, xs_hbm.at[d1_v], sem)
        c0.wait()
        c1.wait()

    return dispatch


def _make_combine():
    mesh = plsc.VectorSubcoreMesh(core_axis_name="c", subcore_axis_name="s")

    @functools.partial(
        pl.kernel, mesh=mesh,
        out_type=jax.ShapeDtypeStruct((TH, DIM), jnp.float32),
        scratch_types=[
            pltpu.VMEM((TWH,), jnp.int32),
            pltpu.VMEM((TWH,), jnp.int32),
            pltpu.VMEM((TWH, L), jnp.float32),
            pltpu.VMEM((TWH, L), jnp.float32),
            pltpu.VMEM((TWH, DIM), jnp.float32),
            pltpu.VMEM((TWH, DIM), jnp.float32),
            pltpu.SemaphoreType.DMA,
        ],
    )
    def combine(d0_hbm, d1_hbm, w0_hbm, w1_hbm, y_hbm, out_hbm,
                d0_v, d1_v, w0_v, w1_v, a_v, b_v, sem):
        wid = lax.axis_index("s") * NC + lax.axis_index("c")
        base = wid * TWH
        pltpu.sync_copy(d0_hbm.at[pl.ds(base, TWH)], d0_v)
        pltpu.sync_copy(d1_hbm.at[pl.ds(base, TWH)], d1_v)
        pltpu.sync_copy(w0_hbm.at[pl.ds(base, TWH)], w0_v)
        pltpu.sync_copy(w1_hbm.at[pl.ds(base, TWH)], w1_v)
        c0 = pltpu.async_copy(y_hbm.at[d0_v], a_v, sem)
        c1 = pltpu.async_copy(y_hbm.at[d1_v], b_v, sem)
        c0.wait()
        c1.wait()

        def body(r, _):
            w0 = w0_v[r, :]
            w1 = w1_v[r, :]
            for c in range(DIM // L):
                sl = pl.ds(c * L, L)
                a_v[r, sl] = a_v[r, sl] * w0 + b_v[r, sl] * w1
            return 0

        lax.fori_loop(0, TWH, body, 0)
        pltpu.sync_copy(a_v, out_hbm.at[pl.ds(base, TWH)])

    return combine


_SC_KERNELS = {}


def _sc_kernels():
    if "combine" not in _SC_KERNELS:
        _SC_KERNELS["dispatch"] = [_make_dispatch(h) for h in range(H)]
        _SC_KERNELS["combine"] = _make_combine()
    return _SC_KERNELS["dispatch"], _SC_KERNELS["combine"]


def kernel(x, Wr, W1, b1, W2, b2, W3, b3):
    dispatches, combine = _sc_kernels()
    xf = x.reshape(T, DIM)

    outs = []
    for h in range(H):
        d0, d1, w0b, w1b, tile_e = _run_meta(xf, Wr, h)
        d0 = d0.reshape(TH)
        d1 = d1.reshape(TH)
        xs = dispatches[h](d0, d1, xf)
        y = _run_gemm(tile_e.reshape(NTH + 1), xs, W1, b1, W2, b2, W3, b3)
        outs.append(combine(d0, d1, w0b, w1b, y))

    return jnp.concatenate(outs, axis=0).reshape(B, S, DIM)


# speedup vs baseline: 1.2596x; 1.2596x over previous
import functools

import jax
import jax.numpy as jnp
from jax import lax
from jax.experimental import pallas as pl
from jax.experimental.pallas import tpu as pltpu
from jax.experimental.pallas import tpu_sc as plsc

B, S, DIM = 1, 2048, 768
FFN = int(DIM * 2.0)
E, K = 8, 2
T = B * S
BT = 256
NI = T // BT
BLK = 256
NTILES = (T * K + E * (BLK - 1) + BLK - 1) // BLK
P = NTILES * BLK

NC, NS, L = 2, 16, 16
NW = NC * NS
TW = T // NW


def _meta_kernel(x_ref, wr_ref, d0_ref, d1_ref, w0_ref, w1_ref, tile_e_ref,
                 carry_ref, meta_ref, base_ref):
    ph = pl.program_id(0)
    i = pl.program_id(1)
    cols = pl.ds(i * BT, BT)
    srow = lax.broadcasted_iota(jnp.int32, (E, BT), 0)

    @pl.when(ph == 0)
    def _phase0():
        x_t = x_ref[...]
        logits_tm = jnp.dot(x_t, wr_ref[...],
                            preferred_element_type=jnp.float32)
        logits = jnp.transpose(logits_tm)
        m = jnp.max(logits, axis=0, keepdims=True)
        ex = jnp.exp(logits - m)
        probs = ex / jnp.sum(ex, axis=0, keepdims=True)
        v1 = jnp.max(probs, axis=0, keepdims=True)
        i1 = jnp.min(jnp.where(probs == v1, srow, E), axis=0, keepdims=True)
        mask1 = srow == i1
        probs2 = jnp.where(mask1, -jnp.inf, probs)
        v2 = jnp.max(probs2, axis=0, keepdims=True)
        i2 = jnp.min(jnp.where(probs2 == v2, srow, E), axis=0, keepdims=True)
        mask2 = srow == i2
        onehot = (mask1 | mask2).astype(jnp.float32)

        @pl.when(i == 0)
        def _init():
            carry_ref[...] = jnp.zeros_like(carry_ref)

        ri = lax.broadcasted_iota(jnp.int32, (BT, BT), 0)
        cj = lax.broadcasted_iota(jnp.int32, (BT, BT), 1)
        utri = (ri < cj).astype(jnp.float32)
        cex = jnp.dot(onehot, utri, preferred_element_type=jnp.float32)
        cex = cex + carry_ref[...]
        carry_ref[...] += jnp.sum(onehot, axis=1, keepdims=True)

        r0 = jnp.sum(jnp.where(mask1, cex, 0.0), axis=0, keepdims=True)
        r1 = jnp.sum(jnp.where(mask2, cex, 0.0), axis=0, keepdims=True)
        meta_ref[:, cols] = jnp.concatenate(
            [r0, r1, i1.astype(jnp.float32), i2.astype(jnp.float32), v1, v2,
             jnp.zeros((2, BT), jnp.float32)], axis=0)

    @pl.when(ph == 1)
    def _phase1():
        @pl.when(i == 0)
        def _bases():
            c = carry_ref[...]
            pc = jnp.floor((c + (BLK - 1)) / BLK) * BLK
            eA = lax.broadcasted_iota(jnp.int32, (E, E), 0)
            eB = lax.broadcasted_iota(jnp.int32, (E, E), 1)
            ltri = (eB < eA).astype(jnp.float32)
            base_ref[...] = jnp.dot(ltri, pc,
                                    preferred_element_type=jnp.float32)
            total = jnp.sum(pc)
            mm = lax.broadcasted_iota(jnp.int32, (E, NTILES + 1), 1) * BLK
            mmc = jnp.minimum(mm.astype(jnp.float32), total - BLK)
            cmp = (jnp.broadcast_to(base_ref[...], (E, NTILES + 1)) <= mmc
                   ).astype(jnp.float32)
            te_raw = jnp.sum(cmp, axis=0, keepdims=True) - 1.0
            mcol = lax.broadcasted_iota(jnp.int32, (1, NTILES + 1), 1)
            te = jnp.where(mcol == NTILES, total * (1.0 / BLK), te_raw)
            tile_e_ref[...] = te.astype(jnp.int32)

        slab = meta_ref[:, cols]

        def getr(c):
            return jnp.sum(jnp.where(srow == c, slab, 0.0), axis=0,
                           keepdims=True)

        r0, r1 = getr(0), getr(1)
        i1, i2 = getr(2).astype(jnp.int32), getr(3).astype(jnp.int32)
        v1, v2 = getr(4), getr(5)
        baseb = jnp.broadcast_to(base_ref[...], (E, BT))
        b0 = jnp.sum(jnp.where(srow == i1, baseb, 0.0), axis=0, keepdims=True)
        b1 = jnp.sum(jnp.where(srow == i2, baseb, 0.0), axis=0, keepdims=True)
        d0_ref[...] = (b0 + r0).astype(jnp.int32).reshape(1, 1, BT)
        d1_ref[...] = (b1 + r1).astype(jnp.int32).reshape(1, 1, BT)
        mrows = jnp.concatenate([v1, v2], axis=0)
        tcol = jnp.transpose(mrows)
        col2 = lax.broadcasted_iota(jnp.int32, tcol.shape, 1)

        def getcol(c):
            return jnp.sum(jnp.where(col2 == c, tcol, 0.0), axis=1,
                           keepdims=True)

        w0_ref[...] = jnp.broadcast_to(getcol(0), (BT, L))
        w1_ref[...] = jnp.broadcast_to(getcol(1), (BT, L))


def _run_meta(xf, Wr):
    return pl.pallas_call(
        _meta_kernel,
        grid=(2, NI),
        in_specs=[
            pl.BlockSpec((BT, DIM), lambda p, i: ((1 - p) * i, 0)),
            pl.BlockSpec((DIM, E), lambda p, i: (0, 0)),
        ],
        out_specs=[
            pl.BlockSpec((1, 1, BT), lambda p, i: (i, 0, 0)),
            pl.BlockSpec((1, 1, BT), lambda p, i: (i, 0, 0)),
            pl.BlockSpec((BT, L), lambda p, i: (i, 0)),
            pl.BlockSpec((BT, L), lambda p, i: (i, 0)),
            pl.BlockSpec((1, NTILES + 1), lambda p, i: (0, 0)),
        ],
        out_shape=[
            jax.ShapeDtypeStruct((NI, 1, BT), jnp.int32),
            jax.ShapeDtypeStruct((NI, 1, BT), jnp.int32),
            jax.ShapeDtypeStruct((T, L), jnp.float32),
            jax.ShapeDtypeStruct((T, L), jnp.float32),
            jax.ShapeDtypeStruct((1, NTILES + 1), jnp.int32),
        ],
        scratch_shapes=[
            pltpu.VMEM((E, 1), jnp.float32),
            pltpu.VMEM((E, T), jnp.float32),
            pltpu.VMEM((E, 1), jnp.float32),
        ],
        compiler_params=pltpu.CompilerParams(
            dimension_semantics=("arbitrary", "arbitrary"),
        ),
    )(xf, Wr)


def _gemm_kernel(te_ref, xs_ref, w1_ref, b1_ref, w2_ref, b2_ref,
                 w3_ref, b3_ref, out_ref):
    m = pl.program_id(0)

    @pl.when(m < te_ref[NTILES])
    def _compute():
        x_t = xs_ref[...]
        h1 = jnp.dot(x_t, w1_ref[0],
                     preferred_element_type=jnp.float32) + b1_ref[0]
        h2 = jnp.dot(x_t, w2_ref[0],
                     preferred_element_type=jnp.float32) + b2_ref[0]
        h = h1 * (1.0 / (1.0 + jnp.exp(-h1))) * h2
        y = jnp.dot(h, w3_ref[0],
                    preferred_element_type=jnp.float32) + b3_ref[0]
        out_ref[...] = y


def _run_gemm(tile_e, xs, W1, b1, W2, b2, W3, b3):
    grid_spec = pltpu.PrefetchScalarGridSpec(
        num_scalar_prefetch=1,
        grid=(NTILES,),
        in_specs=[
            pl.BlockSpec((BLK, DIM), lambda m, te: (m, 0)),
            pl.BlockSpec((1, DIM, FFN), lambda m, te: (te[m], 0, 0)),
            pl.BlockSpec((1, 1, FFN), lambda m, te: (te[m], 0, 0)),
            pl.BlockSpec((1, DIM, FFN), lambda m, te: (te[m], 0, 0)),
            pl.BlockSpec((1, 1, FFN), lambda m, te: (te[m], 0, 0)),
            pl.BlockSpec((1, FFN, DIM), lambda m, te: (te[m], 0, 0)),
            pl.BlockSpec((1, 1, DIM), lambda m, te: (te[m], 0, 0)),
        ],
        out_specs=pl.BlockSpec((BLK, DIM), lambda m, te: (m, 0)),
    )
    return pl.pallas_call(
        _gemm_kernel,
        grid_spec=grid_spec,
        out_shape=jax.ShapeDtypeStruct((P, DIM), jnp.float32),
        compiler_params=pltpu.CompilerParams(
            dimension_semantics=("arbitrary",),
        ),
    )(tile_e, xs, W1, b1[:, None, :], W2, b2[:, None, :], W3, b3[:, None, :])


def _make_dispatch():
    mesh = plsc.VectorSubcoreMesh(core_axis_name="c", subcore_axis_name="s")

    @functools.partial(
        pl.kernel, mesh=mesh,
        out_type=jax.ShapeDtypeStruct((P, DIM), jnp.float32),
        scratch_types=[
            pltpu.VMEM((TW,), jnp.int32),
            pltpu.VMEM((TW,), jnp.int32),
            pltpu.VMEM((TW, DIM), jnp.float32),
            pltpu.SemaphoreType.DMA,
        ],
    )
    def dispatch(d0_hbm, d1_hbm, x_hbm, xs_hbm, d0_v, d1_v, rows_v, sem):
        wid = lax.axis_index("s") * NC + lax.axis_index("c")
        base = wid * TW
        pltpu.sync_copy(d0_hbm.at[pl.ds(base, TW)], d0_v)
        pltpu.sync_copy(d1_hbm.at[pl.ds(base, TW)], d1_v)
        pltpu.sync_copy(x_hbm.at[pl.ds(base, TW)], rows_v)
        c0 = pltpu.async_copy(rows_v, xs_hbm.at[d0_v], sem)
        c1 = pltpu.async_copy(rows_v, xs_hbm.at[d1_v], sem)
        c0.wait()
        c1.wait()

    return dispatch


def _make_combine():
    mesh = plsc.VectorSubcoreMesh(core_axis_name="c", subcore_axis_name="s")

    @functools.partial(
        pl.kernel, mesh=mesh,
        out_type=jax.ShapeDtypeStruct((T, DIM), jnp.float32),
        scratch_types=[
            pltpu.VMEM((TW,), jnp.int32),
            pltpu.VMEM((TW,), jnp.int32),
            pltpu.VMEM((TW, L), jnp.float32),
            pltpu.VMEM((TW, L), jnp.float32),
            pltpu.VMEM((TW, DIM), jnp.float32),
            pltpu.VMEM((TW, DIM), jnp.float32),
            pltpu.SemaphoreType.DMA,
        ],
    )
    def combine(d0_hbm, d1_hbm, w0_hbm, w1_hbm, y_hbm, out_hbm,
                d0_v, d1_v, w0_v, w1_v, a_v, b_v, sem):
        wid = lax.axis_index("s") * NC + lax.axis_index("c")
        base = wid * TW
        pltpu.sync_copy(d0_hbm.at[pl.ds(base, TW)], d0_v)
        pltpu.sync_copy(d1_hbm.at[pl.ds(base, TW)], d1_v)
        pltpu.sync_copy(w0_hbm.at[pl.ds(base, TW)], w0_v)
        pltpu.sync_copy(w1_hbm.at[pl.ds(base, TW)], w1_v)
        hw = TW // 2
        cp = [None] * 4
        for s in range(2):
            ts = pl.ds(s * hw, hw)
            cp[2 * s] = pltpu.async_copy(
                y_hbm.at[d0_v.at[ts]], a_v.at[ts, :], sem)
            cp[2 * s + 1] = pltpu.async_copy(
                y_hbm.at[d1_v.at[ts]], b_v.at[ts, :], sem)

        def body(r, _):
            w0 = w0_v[r, :]
            w1 = w1_v[r, :]
            for c in range(DIM // L):
                sl = pl.ds(c * L, L)
                a_v[r, sl] = a_v[r, sl] * w0 + b_v[r, sl] * w1
            return 0

        cp[0].wait()
        cp[1].wait()
        lax.fori_loop(0, hw, body, 0)
        cp[2].wait()
        cp[3].wait()
        lax.fori_loop(hw, TW, body, 0)
        pltpu.sync_copy(a_v, out_hbm.at[pl.ds(base, TW)])

    return combine


_SC_KERNELS = {}


def _sc_kernels():
    if "dispatch" not in _SC_KERNELS:
        _SC_KERNELS["dispatch"] = _make_dispatch()
        _SC_KERNELS["combine"] = _make_combine()
    return _SC_KERNELS["dispatch"], _SC_KERNELS["combine"]


def kernel(x, Wr, W1, b1, W2, b2, W3, b3):
    dispatch, combine = _sc_kernels()
    xf = x.reshape(T, DIM)

    d0, d1, w0b, w1b, tile_e = _run_meta(xf, Wr)
    d0 = d0.reshape(T)
    d1 = d1.reshape(T)

    xs = dispatch(d0, d1, xf)

    y = _run_gemm(tile_e.reshape(NTILES + 1), xs, W1, b1, W2, b2, W3, b3)

    out = combine(d0, d1, w0b, w1b, y)

    return out.reshape(B, S, DIM)


# speedup vs baseline: 1.3571x; 1.0774x over previous
import functools

import jax
import jax.numpy as jnp
from jax import lax
from jax.experimental import pallas as pl
from jax.experimental.pallas import tpu as pltpu
from jax.experimental.pallas import tpu_sc as plsc

B, S, DIM = 1, 2048, 768
FFN = int(DIM * 2.0)
E, K = 8, 2
T = B * S
BT = 256
NI = T // BT
BLK = 512
NTILES = (T * K + E * (BLK - 1) + BLK - 1) // BLK
P = NTILES * BLK

NC, NS, L = 2, 16, 16
NW = NC * NS
TW = T // NW


def _meta_kernel(x_ref, wr_ref, d0_ref, d1_ref, w0_ref, w1_ref, tile_e_ref,
                 carry_ref, meta_ref, base_ref):
    ph = pl.program_id(0)
    i = pl.program_id(1)
    cols = pl.ds(i * BT, BT)
    srow = lax.broadcasted_iota(jnp.int32, (E, BT), 0)

    @pl.when(ph == 0)
    def _phase0():
        x_t = x_ref[...]
        logits_tm = jnp.dot(x_t, wr_ref[...],
                            preferred_element_type=jnp.float32)
        logits = jnp.transpose(logits_tm)
        m = jnp.max(logits, axis=0, keepdims=True)
        ex = jnp.exp(logits - m)
        probs = ex / jnp.sum(ex, axis=0, keepdims=True)
        v1 = jnp.max(probs, axis=0, keepdims=True)
        i1 = jnp.min(jnp.where(probs == v1, srow, E), axis=0, keepdims=True)
        mask1 = srow == i1
        probs2 = jnp.where(mask1, -jnp.inf, probs)
        v2 = jnp.max(probs2, axis=0, keepdims=True)
        i2 = jnp.min(jnp.where(probs2 == v2, srow, E), axis=0, keepdims=True)
        mask2 = srow == i2
        onehot = (mask1 | mask2).astype(jnp.float32)

        @pl.when(i == 0)
        def _init():
            carry_ref[...] = jnp.zeros_like(carry_ref)

        ri = lax.broadcasted_iota(jnp.int32, (BT, BT), 0)
        cj = lax.broadcasted_iota(jnp.int32, (BT, BT), 1)
        utri = (ri < cj).astype(jnp.float32)
        cex = jnp.dot(onehot, utri, preferred_element_type=jnp.float32)
        cex = cex + carry_ref[...]
        carry_ref[...] += jnp.sum(onehot, axis=1, keepdims=True)

        r0 = jnp.sum(jnp.where(mask1, cex, 0.0), axis=0, keepdims=True)
        r1 = jnp.sum(jnp.where(mask2, cex, 0.0), axis=0, keepdims=True)
        meta_ref[:, cols] = jnp.concatenate(
            [r0, r1, i1.astype(jnp.float32), i2.astype(jnp.float32), v1, v2,
             jnp.zeros((2, BT), jnp.float32)], axis=0)

    @pl.when(ph == 1)
    def _phase1():
        @pl.when(i == 0)
        def _bases():
            c = carry_ref[...]
            pc = jnp.floor((c + (BLK - 1)) / BLK) * BLK
            eA = lax.broadcasted_iota(jnp.int32, (E, E), 0)
            eB = lax.broadcasted_iota(jnp.int32, (E, E), 1)
            ltri = (eB < eA).astype(jnp.float32)
            base_ref[...] = jnp.dot(ltri, pc,
                                    preferred_element_type=jnp.float32)
            total = jnp.sum(pc)
            mm = lax.broadcasted_iota(jnp.int32, (E, NTILES + 1), 1) * BLK
            mmc = jnp.minimum(mm.astype(jnp.float32), total - BLK)
            cmp = (jnp.broadcast_to(base_ref[...], (E, NTILES + 1)) <= mmc
                   ).astype(jnp.float32)
            te_raw = jnp.sum(cmp, axis=0, keepdims=True) - 1.0
            mcol = lax.broadcasted_iota(jnp.int32, (1, NTILES + 1), 1)
            te = jnp.where(mcol == NTILES, total * (1.0 / BLK), te_raw)
            tile_e_ref[...] = te.astype(jnp.int32)

        slab = meta_ref[:, cols]

        def getr(c):
            return jnp.sum(jnp.where(srow == c, slab, 0.0), axis=0,
                           keepdims=True)

        r0, r1 = getr(0), getr(1)
        i1, i2 = getr(2).astype(jnp.int32), getr(3).astype(jnp.int32)
        v1, v2 = getr(4), getr(5)
        baseb = jnp.broadcast_to(base_ref[...], (E, BT))
        b0 = jnp.sum(jnp.where(srow == i1, baseb, 0.0), axis=0, keepdims=True)
        b1 = jnp.sum(jnp.where(srow == i2, baseb, 0.0), axis=0, keepdims=True)
        d0_ref[...] = (b0 + r0).astype(jnp.int32).reshape(1, 1, BT)
        d1_ref[...] = (b1 + r1).astype(jnp.int32).reshape(1, 1, BT)
        mrows = jnp.concatenate([v1, v2], axis=0)
        tcol = jnp.transpose(mrows)
        col2 = lax.broadcasted_iota(jnp.int32, tcol.shape, 1)

        def getcol(c):
            return jnp.sum(jnp.where(col2 == c, tcol, 0.0), axis=1,
                           keepdims=True)

        w0_ref[...] = jnp.broadcast_to(getcol(0), (BT, L))
        w1_ref[...] = jnp.broadcast_to(getcol(1), (BT, L))


def _run_meta(xf, Wr):
    return pl.pallas_call(
        _meta_kernel,
        grid=(2, NI),
        in_specs=[
            pl.BlockSpec((BT, DIM), lambda p, i: ((1 - p) * i, 0)),
            pl.BlockSpec((DIM, E), lambda p, i: (0, 0)),
        ],
        out_specs=[
            pl.BlockSpec((1, 1, BT), lambda p, i: (i, 0, 0)),
            pl.BlockSpec((1, 1, BT), lambda p, i: (i, 0, 0)),
            pl.BlockSpec((BT, L), lambda p, i: (i, 0)),
            pl.BlockSpec((BT, L), lambda p, i: (i, 0)),
            pl.BlockSpec((1, NTILES + 1), lambda p, i: (0, 0)),
        ],
        out_shape=[
            jax.ShapeDtypeStruct((NI, 1, BT), jnp.int32),
            jax.ShapeDtypeStruct((NI, 1, BT), jnp.int32),
            jax.ShapeDtypeStruct((T, L), jnp.float32),
            jax.ShapeDtypeStruct((T, L), jnp.float32),
            jax.ShapeDtypeStruct((1, NTILES + 1), jnp.int32),
        ],
        scratch_shapes=[
            pltpu.VMEM((E, 1), jnp.float32),
            pltpu.VMEM((E, T), jnp.float32),
            pltpu.VMEM((E, 1), jnp.float32),
        ],
        compiler_params=pltpu.CompilerParams(
            dimension_semantics=("arbitrary", "arbitrary"),
        ),
    )(xf, Wr)


def _gemm_kernel(te_ref, xs_ref, w1_ref, b1_ref, w2_ref, b2_ref,
                 w3_ref, b3_ref, out_ref):
    m = pl.program_id(0)

    @pl.when(m < te_ref[NTILES])
    def _compute():
        x_t = xs_ref[...]
        h1 = jnp.dot(x_t, w1_ref[0],
                     preferred_element_type=jnp.float32) + b1_ref[0]
        h2 = jnp.dot(x_t, w2_ref[0],
                     preferred_element_type=jnp.float32) + b2_ref[0]
        h = h1 * (1.0 / (1.0 + jnp.exp(-h1))) * h2
        y = jnp.dot(h, w3_ref[0],
                    preferred_element_type=jnp.float32) + b3_ref[0]
        out_ref[...] = y


def _run_gemm(tile_e, xs, W1, b1, W2, b2, W3, b3):
    grid_spec = pltpu.PrefetchScalarGridSpec(
        num_scalar_prefetch=1,
        grid=(NTILES,),
        in_specs=[
            pl.BlockSpec((BLK, DIM), lambda m, te: (m, 0)),
            pl.BlockSpec((1, DIM, FFN), lambda m, te: (te[m], 0, 0)),
            pl.BlockSpec((1, 1, FFN), lambda m, te: (te[m], 0, 0)),
            pl.BlockSpec((1, DIM, FFN), lambda m, te: (te[m], 0, 0)),
            pl.BlockSpec((1, 1, FFN), lambda m, te: (te[m], 0, 0)),
            pl.BlockSpec((1, FFN, DIM), lambda m, te: (te[m], 0, 0)),
            pl.BlockSpec((1, 1, DIM), lambda m, te: (te[m], 0, 0)),
        ],
        out_specs=pl.BlockSpec((BLK, DIM), lambda m, te: (m, 0)),
    )
    return pl.pallas_call(
        _gemm_kernel,
        grid_spec=grid_spec,
        out_shape=jax.ShapeDtypeStruct((P, DIM), jnp.float32),
        compiler_params=pltpu.CompilerParams(
            dimension_semantics=("arbitrary",),
        ),
    )(tile_e, xs, W1, b1[:, None, :], W2, b2[:, None, :], W3, b3[:, None, :])


def _make_dispatch():
    mesh = plsc.VectorSubcoreMesh(core_axis_name="c", subcore_axis_name="s")

    @functools.partial(
        pl.kernel, mesh=mesh,
        out_type=jax.ShapeDtypeStruct((P, DIM), jnp.float32),
        scratch_types=[
            pltpu.VMEM((TW,), jnp.int32),
            pltpu.VMEM((TW,), jnp.int32),
            pltpu.VMEM((TW, DIM), jnp.float32),
            pltpu.SemaphoreType.DMA,
        ],
    )
    def dispatch(d0_hbm, d1_hbm, x_hbm, xs_hbm, d0_v, d1_v, rows_v, sem):
        wid = lax.axis_index("s") * NC + lax.axis_index("c")
        base = wid * TW
        pltpu.sync_copy(d0_hbm.at[pl.ds(base, TW)], d0_v)
        pltpu.sync_copy(d1_hbm.at[pl.ds(base, TW)], d1_v)
        pltpu.sync_copy(x_hbm.at[pl.ds(base, TW)], rows_v)
        c0 = pltpu.async_copy(rows_v, xs_hbm.at[d0_v], sem)
        c1 = pltpu.async_copy(rows_v, xs_hbm.at[d1_v], sem)
        c0.wait()
        c1.wait()

    return dispatch


def _make_combine():
    mesh = plsc.VectorSubcoreMesh(core_axis_name="c", subcore_axis_name="s")

    @functools.partial(
        pl.kernel, mesh=mesh,
        out_type=jax.ShapeDtypeStruct((T, DIM), jnp.float32),
        scratch_types=[
            pltpu.VMEM((TW,), jnp.int32),
            pltpu.VMEM((TW,), jnp.int32),
            pltpu.VMEM((TW, L), jnp.float32),
            pltpu.VMEM((TW, L), jnp.float32),
            pltpu.VMEM((TW, DIM), jnp.float32),
            pltpu.VMEM((TW, DIM), jnp.float32),
            pltpu.SemaphoreType.DMA,
        ],
    )
    def combine(d0_hbm, d1_hbm, w0_hbm, w1_hbm, y_hbm, out_hbm,
                d0_v, d1_v, w0_v, w1_v, a_v, b_v, sem):
        wid = lax.axis_index("s") * NC + lax.axis_index("c")
        base = wid * TW
        pltpu.sync_copy(d0_hbm.at[pl.ds(base, TW)], d0_v)
        pltpu.sync_copy(d1_hbm.at[pl.ds(base, TW)], d1_v)
        pltpu.sync_copy(w0_hbm.at[pl.ds(base, TW)], w0_v)
        pltpu.sync_copy(w1_hbm.at[pl.ds(base, TW)], w1_v)
        hw = TW // 2
        cp = [None] * 4
        for s in range(2):
            ts = pl.ds(s * hw, hw)
            cp[2 * s] = pltpu.async_copy(
                y_hbm.at[d0_v.at[ts]], a_v.at[ts, :], sem)
            cp[2 * s + 1] = pltpu.async_copy(
                y_hbm.at[d1_v.at[ts]], b_v.at[ts, :], sem)

        def body(r, _):
            w0 = w0_v[r, :]
            w1 = w1_v[r, :]
            for c in range(DIM // L):
                sl = pl.ds(c * L, L)
                a_v[r, sl] = a_v[r, sl] * w0 + b_v[r, sl] * w1
            return 0

        cp[0].wait()
        cp[1].wait()
        lax.fori_loop(0, hw, body, 0)
        cp[2].wait()
        cp[3].wait()
        lax.fori_loop(hw, TW, body, 0)
        pltpu.sync_copy(a_v, out_hbm.at[pl.ds(base, TW)])

    return combine


_SC_KERNELS = {}


def _sc_kernels():
    if "dispatch" not in _SC_KERNELS:
        _SC_KERNELS["dispatch"] = _make_dispatch()
        _SC_KERNELS["combine"] = _make_combine()
    return _SC_KERNELS["dispatch"], _SC_KERNELS["combine"]


def kernel(x, Wr, W1, b1, W2, b2, W3, b3):
    dispatch, combine = _sc_kernels()
    xf = x.reshape(T, DIM)

    d0, d1, w0b, w1b, tile_e = _run_meta(xf, Wr)
    d0 = d0.reshape(T)
    d1 = d1.reshape(T)

    xs = dispatch(d0, d1, xf)

    y = _run_gemm(tile_e.reshape(NTILES + 1), xs, W1, b1, W2, b2, W3, b3)

    out = combine(d0, d1, w0b, w1b, y)

    return out.reshape(B, S, DIM)


# speedup vs baseline: 1.3997x; 1.0314x over previous
import functools

import jax
import jax.numpy as jnp
from jax import lax
from jax.experimental import pallas as pl
from jax.experimental.pallas import tpu as pltpu
from jax.experimental.pallas import tpu_sc as plsc

B, S, DIM = 1, 2048, 768
FFN = int(DIM * 2.0)
E, K = 8, 2
T = B * S
BT = 512
NI = T // BT
BLK = 512
NTILES = (T * K + E * (BLK - 1) + BLK - 1) // BLK
P = NTILES * BLK

NC, NS, L = 2, 16, 16
NW = NC * NS
TW = T // NW


def _meta_kernel(x_ref, wr_ref, d0_ref, d1_ref, w0_ref, w1_ref, tile_e_ref,
                 carry_ref, meta_ref, base_ref):
    ph = pl.program_id(0)
    i = pl.program_id(1)
    cols = pl.ds(i * BT, BT)
    srow = lax.broadcasted_iota(jnp.int32, (E, BT), 0)

    @pl.when(ph == 0)
    def _phase0():
        x_t = x_ref[...]
        logits_tm = jnp.dot(x_t, wr_ref[...],
                            preferred_element_type=jnp.float32)
        logits = jnp.transpose(logits_tm)
        m = jnp.max(logits, axis=0, keepdims=True)
        ex = jnp.exp(logits - m)
        probs = ex / jnp.sum(ex, axis=0, keepdims=True)
        v1 = jnp.max(probs, axis=0, keepdims=True)
        i1 = jnp.min(jnp.where(probs == v1, srow, E), axis=0, keepdims=True)
        mask1 = srow == i1
        probs2 = jnp.where(mask1, -jnp.inf, probs)
        v2 = jnp.max(probs2, axis=0, keepdims=True)
        i2 = jnp.min(jnp.where(probs2 == v2, srow, E), axis=0, keepdims=True)
        mask2 = srow == i2
        onehot = (mask1 | mask2).astype(jnp.float32)

        @pl.when(i == 0)
        def _init():
            carry_ref[...] = jnp.zeros_like(carry_ref)

        ri = lax.broadcasted_iota(jnp.int32, (BT, BT), 0)
        cj = lax.broadcasted_iota(jnp.int32, (BT, BT), 1)
        utri = (ri < cj).astype(jnp.float32)
        cex = jnp.dot(onehot, utri, preferred_element_type=jnp.float32)
        cex = cex + carry_ref[...]
        carry_ref[...] += jnp.sum(onehot, axis=1, keepdims=True)

        r0 = jnp.sum(jnp.where(mask1, cex, 0.0), axis=0, keepdims=True)
        r1 = jnp.sum(jnp.where(mask2, cex, 0.0), axis=0, keepdims=True)
        meta_ref[:, cols] = jnp.concatenate(
            [r0, r1, i1.astype(jnp.float32), i2.astype(jnp.float32), v1, v2,
             jnp.zeros((2, BT), jnp.float32)], axis=0)

    @pl.when(ph == 1)
    def _phase1():
        @pl.when(i == 0)
        def _bases():
            c = carry_ref[...]
            pc = jnp.floor((c + (BLK - 1)) / BLK) * BLK
            eA = lax.broadcasted_iota(jnp.int32, (E, E), 0)
            eB = lax.broadcasted_iota(jnp.int32, (E, E), 1)
            ltri = (eB < eA).astype(jnp.float32)
            base_ref[...] = jnp.dot(ltri, pc,
                                    preferred_element_type=jnp.float32)
            total = jnp.sum(pc)
            mm = lax.broadcasted_iota(jnp.int32, (E, NTILES + 1), 1) * BLK
            mmc = jnp.minimum(mm.astype(jnp.float32), total - BLK)
            cmp = (jnp.broadcast_to(base_ref[...], (E, NTILES + 1)) <= mmc
                   ).astype(jnp.float32)
            te_raw = jnp.sum(cmp, axis=0, keepdims=True) - 1.0
            mcol = lax.broadcasted_iota(jnp.int32, (1, NTILES + 1), 1)
            te = jnp.where(mcol == NTILES, total * (1.0 / BLK), te_raw)
            tile_e_ref[...] = te.astype(jnp.int32)

        slab = meta_ref[:, cols]

        def getr(c):
            return jnp.sum(jnp.where(srow == c, slab, 0.0), axis=0,
                           keepdims=True)

        r0, r1 = getr(0), getr(1)
        i1, i2 = getr(2).astype(jnp.int32), getr(3).astype(jnp.int32)
        v1, v2 = getr(4), getr(5)
        baseb = jnp.broadcast_to(base_ref[...], (E, BT))
        b0 = jnp.sum(jnp.where(srow == i1, baseb, 0.0), axis=0, keepdims=True)
        b1 = jnp.sum(jnp.where(srow == i2, baseb, 0.0), axis=0, keepdims=True)
        d0_ref[...] = (b0 + r0).astype(jnp.int32).reshape(1, 1, BT)
        d1_ref[...] = (b1 + r1).astype(jnp.int32).reshape(1, 1, BT)
        mrows = jnp.concatenate([v1, v2], axis=0)
        tcol = jnp.transpose(mrows)
        col2 = lax.broadcasted_iota(jnp.int32, tcol.shape, 1)

        def getcol(c):
            return jnp.sum(jnp.where(col2 == c, tcol, 0.0), axis=1,
                           keepdims=True)

        w0_ref[...] = jnp.broadcast_to(getcol(0), (BT, L))
        w1_ref[...] = jnp.broadcast_to(getcol(1), (BT, L))


def _run_meta(xf, Wr):
    return pl.pallas_call(
        _meta_kernel,
        grid=(2, NI),
        in_specs=[
            pl.BlockSpec((BT, DIM), lambda p, i: ((1 - p) * i, 0)),
            pl.BlockSpec((DIM, E), lambda p, i: (0, 0)),
        ],
        out_specs=[
            pl.BlockSpec((1, 1, BT), lambda p, i: (i, 0, 0)),
            pl.BlockSpec((1, 1, BT), lambda p, i: (i, 0, 0)),
            pl.BlockSpec((BT, L), lambda p, i: (i, 0)),
            pl.BlockSpec((BT, L), lambda p, i: (i, 0)),
            pl.BlockSpec((1, NTILES + 1), lambda p, i: (0, 0)),
        ],
        out_shape=[
            jax.ShapeDtypeStruct((NI, 1, BT), jnp.int32),
            jax.ShapeDtypeStruct((NI, 1, BT), jnp.int32),
            jax.ShapeDtypeStruct((T, L), jnp.float32),
            jax.ShapeDtypeStruct((T, L), jnp.float32),
            jax.ShapeDtypeStruct((1, NTILES + 1), jnp.int32),
        ],
        scratch_shapes=[
            pltpu.VMEM((E, 1), jnp.float32),
            pltpu.VMEM((E, T), jnp.float32),
            pltpu.VMEM((E, 1), jnp.float32),
        ],
        compiler_params=pltpu.CompilerParams(
            dimension_semantics=("arbitrary", "arbitrary"),
        ),
    )(xf, Wr)


def _gemm_kernel(te_ref, xs_ref, w1_ref, b1_ref, w2_ref, b2_ref,
                 w3_ref, b3_ref, out_ref):
    m = pl.program_id(0)

    @pl.when(m < te_ref[NTILES])
    def _compute():
        x_t = xs_ref[...]
        h1 = jnp.dot(x_t, w1_ref[0],
                     preferred_element_type=jnp.float32) + b1_ref[0]
        h2 = jnp.dot(x_t, w2_ref[0],
                     preferred_element_type=jnp.float32) + b2_ref[0]
        h = h1 * (1.0 / (1.0 + jnp.exp(-h1))) * h2
        y = jnp.dot(h, w3_ref[0],
                    preferred_element_type=jnp.float32) + b3_ref[0]
        out_ref[...] = y


def _run_gemm(tile_e, xs, W1, b1, W2, b2, W3, b3):
    grid_spec = pltpu.PrefetchScalarGridSpec(
        num_scalar_prefetch=1,
        grid=(NTILES,),
        in_specs=[
            pl.BlockSpec((BLK, DIM), lambda m, te: (m, 0)),
            pl.BlockSpec((1, DIM, FFN), lambda m, te: (te[m], 0, 0)),
            pl.BlockSpec((1, 1, FFN), lambda m, te: (te[m], 0, 0)),
            pl.BlockSpec((1, DIM, FFN), lambda m, te: (te[m], 0, 0)),
            pl.BlockSpec((1, 1, FFN), lambda m, te: (te[m], 0, 0)),
            pl.BlockSpec((1, FFN, DIM), lambda m, te: (te[m], 0, 0)),
            pl.BlockSpec((1, 1, DIM), lambda m, te: (te[m], 0, 0)),
        ],
        out_specs=pl.BlockSpec((BLK, DIM), lambda m, te: (m, 0)),
    )
    return pl.pallas_call(
        _gemm_kernel,
        grid_spec=grid_spec,
        out_shape=jax.ShapeDtypeStruct((P, DIM), jnp.float32),
        compiler_params=pltpu.CompilerParams(
            dimension_semantics=("arbitrary",),
        ),
    )(tile_e, xs, W1, b1[:, None, :], W2, b2[:, None, :], W3, b3[:, None, :])


def _make_dispatch():
    mesh = plsc.VectorSubcoreMesh(core_axis_name="c", subcore_axis_name="s")

    @functools.partial(
        pl.kernel, mesh=mesh,
        out_type=jax.ShapeDtypeStruct((P, DIM), jnp.float32),
        scratch_types=[
            pltpu.VMEM((TW,), jnp.int32),
            pltpu.VMEM((TW,), jnp.int32),
            pltpu.VMEM((TW, DIM), jnp.float32),
            pltpu.SemaphoreType.DMA,
        ],
    )
    def dispatch(d0_hbm, d1_hbm, x_hbm, xs_hbm, d0_v, d1_v, rows_v, sem):
        wid = lax.axis_index("s") * NC + lax.axis_index("c")
        base = wid * TW
        pltpu.sync_copy(d0_hbm.at[pl.ds(base, TW)], d0_v)
        pltpu.sync_copy(d1_hbm.at[pl.ds(base, TW)], d1_v)
        pltpu.sync_copy(x_hbm.at[pl.ds(base, TW)], rows_v)
        c0 = pltpu.async_copy(rows_v, xs_hbm.at[d0_v], sem)
        c1 = pltpu.async_copy(rows_v, xs_hbm.at[d1_v], sem)
        c0.wait()
        c1.wait()

    return dispatch


def _make_combine():
    mesh = plsc.VectorSubcoreMesh(core_axis_name="c", subcore_axis_name="s")

    @functools.partial(
        pl.kernel, mesh=mesh,
        out_type=jax.ShapeDtypeStruct((T, DIM), jnp.float32),
        scratch_types=[
            pltpu.VMEM((TW,), jnp.int32),
            pltpu.VMEM((TW,), jnp.int32),
            pltpu.VMEM((TW, L), jnp.float32),
            pltpu.VMEM((TW, L), jnp.float32),
            pltpu.VMEM((TW, DIM), jnp.float32),
            pltpu.VMEM((TW, DIM), jnp.float32),
            pltpu.SemaphoreType.DMA,
        ],
    )
    def combine(d0_hbm, d1_hbm, w0_hbm, w1_hbm, y_hbm, out_hbm,
                d0_v, d1_v, w0_v, w1_v, a_v, b_v, sem):
        wid = lax.axis_index("s") * NC + lax.axis_index("c")
        base = wid * TW
        pltpu.sync_copy(d0_hbm.at[pl.ds(base, TW)], d0_v)
        pltpu.sync_copy(d1_hbm.at[pl.ds(base, TW)], d1_v)
        pltpu.sync_copy(w0_hbm.at[pl.ds(base, TW)], w0_v)
        pltpu.sync_copy(w1_hbm.at[pl.ds(base, TW)], w1_v)
        hw = TW // 2
        cp = [None] * 4
        for s in range(2):
            ts = pl.ds(s * hw, hw)
            cp[2 * s] = pltpu.async_copy(
                y_hbm.at[d0_v.at[ts]], a_v.at[ts, :], sem)
            cp[2 * s + 1] = pltpu.async_copy(
                y_hbm.at[d1_v.at[ts]], b_v.at[ts, :], sem)

        def body(r, _):
            w0 = w0_v[r, :]
            w1 = w1_v[r, :]
            for c in range(DIM // L):
                sl = pl.ds(c * L, L)
                a_v[r, sl] = a_v[r, sl] * w0 + b_v[r, sl] * w1
            return 0

        cp[0].wait()
        cp[1].wait()
        lax.fori_loop(0, hw, body, 0)
        cp[2].wait()
        cp[3].wait()
        lax.fori_loop(hw, TW, body, 0)
        pltpu.sync_copy(a_v, out_hbm.at[pl.ds(base, TW)])

    return combine


_SC_KERNELS = {}


def _sc_kernels():
    if "dispatch" not in _SC_KERNELS:
        _SC_KERNELS["dispatch"] = _make_dispatch()
        _SC_KERNELS["combine"] = _make_combine()
    return _SC_KERNELS["dispatch"], _SC_KERNELS["combine"]


def kernel(x, Wr, W1, b1, W2, b2, W3, b3):
    dispatch, combine = _sc_kernels()
    xf = x.reshape(T, DIM)

    d0, d1, w0b, w1b, tile_e = _run_meta(xf, Wr)
    d0 = d0.reshape(T)
    d1 = d1.reshape(T)

    xs = dispatch(d0, d1, xf)

    y = _run_gemm(tile_e.reshape(NTILES + 1), xs, W1, b1, W2, b2, W3, b3)

    out = combine(d0, d1, w0b, w1b, y)

    return out.reshape(B, S, DIM)


# speedup vs baseline: 1.4161x; 1.0117x over previous
import functools

import jax
import jax.numpy as jnp
from jax import lax
from jax.experimental import pallas as pl
from jax.experimental.pallas import tpu as pltpu
from jax.experimental.pallas import tpu_sc as plsc

B, S, DIM = 1, 2048, 768
FFN = int(DIM * 2.0)
E, K = 8, 2
T = B * S
BT = 1024
NI = T // BT
BLK = 512
NTILES = (T * K + E * (BLK - 1) + BLK - 1) // BLK
P = NTILES * BLK

NC, NS, L = 2, 16, 16
NW = NC * NS
TW = T // NW


def _meta_kernel(x_ref, wr_ref, d0_ref, d1_ref, w0_ref, w1_ref, tile_e_ref,
                 carry_ref, meta_ref, base_ref):
    ph = pl.program_id(0)
    i = pl.program_id(1)
    cols = pl.ds(i * BT, BT)
    srow = lax.broadcasted_iota(jnp.int32, (E, BT), 0)

    @pl.when(ph == 0)
    def _phase0():
        x_t = x_ref[...]
        logits_tm = jnp.dot(x_t, wr_ref[...],
                            preferred_element_type=jnp.float32)
        logits = jnp.transpose(logits_tm)
        m = jnp.max(logits, axis=0, keepdims=True)
        ex = jnp.exp(logits - m)
        probs = ex / jnp.sum(ex, axis=0, keepdims=True)
        v1 = jnp.max(probs, axis=0, keepdims=True)
        i1 = jnp.min(jnp.where(probs == v1, srow, E), axis=0, keepdims=True)
        mask1 = srow == i1
        probs2 = jnp.where(mask1, -jnp.inf, probs)
        v2 = jnp.max(probs2, axis=0, keepdims=True)
        i2 = jnp.min(jnp.where(probs2 == v2, srow, E), axis=0, keepdims=True)
        mask2 = srow == i2
        onehot = (mask1 | mask2).astype(jnp.float32)

        @pl.when(i == 0)
        def _init():
            carry_ref[...] = jnp.zeros_like(carry_ref)

        ri = lax.broadcasted_iota(jnp.int32, (BT, BT), 0)
        cj = lax.broadcasted_iota(jnp.int32, (BT, BT), 1)
        utri = (ri < cj).astype(jnp.float32)
        cex = jnp.dot(onehot, utri, preferred_element_type=jnp.float32)
        cex = cex + carry_ref[...]
        carry_ref[...] += jnp.sum(onehot, axis=1, keepdims=True)

        r0 = jnp.sum(jnp.where(mask1, cex, 0.0), axis=0, keepdims=True)
        r1 = jnp.sum(jnp.where(mask2, cex, 0.0), axis=0, keepdims=True)
        meta_ref[:, cols] = jnp.concatenate(
            [r0, r1, i1.astype(jnp.float32), i2.astype(jnp.float32), v1, v2,
             jnp.zeros((2, BT), jnp.float32)], axis=0)

    @pl.when(ph == 1)
    def _phase1():
        @pl.when(i == 0)
        def _bases():
            c = carry_ref[...]
            pc = jnp.floor((c + (BLK - 1)) / BLK) * BLK
            eA = lax.broadcasted_iota(jnp.int32, (E, E), 0)
            eB = lax.broadcasted_iota(jnp.int32, (E, E), 1)
            ltri = (eB < eA).astype(jnp.float32)
            base_ref[...] = jnp.dot(ltri, pc,
                                    preferred_element_type=jnp.float32)
            total = jnp.sum(pc)
            mm = lax.broadcasted_iota(jnp.int32, (E, NTILES + 1), 1) * BLK
            mmc = jnp.minimum(mm.astype(jnp.float32), total - BLK)
            cmp = (jnp.broadcast_to(base_ref[...], (E, NTILES + 1)) <= mmc
                   ).astype(jnp.float32)
            te_raw = jnp.sum(cmp, axis=0, keepdims=True) - 1.0
            mcol = lax.broadcasted_iota(jnp.int32, (1, NTILES + 1), 1)
            te = jnp.where(mcol == NTILES, total * (1.0 / BLK), te_raw)
            tile_e_ref[...] = te.astype(jnp.int32)

        slab = meta_ref[:, cols]

        def getr(c):
            return jnp.sum(jnp.where(srow == c, slab, 0.0), axis=0,
                           keepdims=True)

        r0, r1 = getr(0), getr(1)
        i1, i2 = getr(2).astype(jnp.int32), getr(3).astype(jnp.int32)
        v1, v2 = getr(4), getr(5)
        baseb = jnp.broadcast_to(base_ref[...], (E, BT))
        b0 = jnp.sum(jnp.where(srow == i1, baseb, 0.0), axis=0, keepdims=True)
        b1 = jnp.sum(jnp.where(srow == i2, baseb, 0.0), axis=0, keepdims=True)
        d0_ref[...] = (b0 + r0).astype(jnp.int32).reshape(1, 1, BT)
        d1_ref[...] = (b1 + r1).astype(jnp.int32).reshape(1, 1, BT)
        mrows = jnp.concatenate([v1, v2], axis=0)
        tcol = jnp.transpose(mrows)
        col2 = lax.broadcasted_iota(jnp.int32, tcol.shape, 1)

        def getcol(c):
            return jnp.sum(jnp.where(col2 == c, tcol, 0.0), axis=1,
                           keepdims=True)

        w0_ref[...] = jnp.broadcast_to(getcol(0), (BT, L))
        w1_ref[...] = jnp.broadcast_to(getcol(1), (BT, L))


def _run_meta(xf, Wr):
    return pl.pallas_call(
        _meta_kernel,
        grid=(2, NI),
        in_specs=[
            pl.BlockSpec((BT, DIM), lambda p, i: ((1 - p) * i, 0)),
            pl.BlockSpec((DIM, E), lambda p, i: (0, 0)),
        ],
        out_specs=[
            pl.BlockSpec((1, 1, BT), lambda p, i: (i, 0, 0)),
            pl.BlockSpec((1, 1, BT), lambda p, i: (i, 0, 0)),
            pl.BlockSpec((BT, L), lambda p, i: (i, 0)),
            pl.BlockSpec((BT, L), lambda p, i: (i, 0)),
            pl.BlockSpec((1, NTILES + 1), lambda p, i: (0, 0)),
        ],
        out_shape=[
            jax.ShapeDtypeStruct((NI, 1, BT), jnp.int32),
            jax.ShapeDtypeStruct((NI, 1, BT), jnp.int32),
            jax.ShapeDtypeStruct((T, L), jnp.float32),
            jax.ShapeDtypeStruct((T, L), jnp.float32),
            jax.ShapeDtypeStruct((1, NTILES + 1), jnp.int32),
        ],
        scratch_shapes=[
            pltpu.VMEM((E, 1), jnp.float32),
            pltpu.VMEM((E, T), jnp.float32),
            pltpu.VMEM((E, 1), jnp.float32),
        ],
        compiler_params=pltpu.CompilerParams(
            dimension_semantics=("arbitrary", "arbitrary"),
        ),
    )(xf, Wr)


def _gemm_kernel(te_ref, xs_ref, w1_ref, b1_ref, w2_ref, b2_ref,
                 w3_ref, b3_ref, out_ref):
    m = pl.program_id(0)

    @pl.when(m < te_ref[NTILES])
    def _compute():
        x_t = xs_ref[...]
        h1 = jnp.dot(x_t, w1_ref[0],
                     preferred_element_type=jnp.float32) + b1_ref[0]
        h2 = jnp.dot(x_t, w2_ref[0],
                     preferred_element_type=jnp.float32) + b2_ref[0]
        h = h1 * (1.0 / (1.0 + jnp.exp(-h1))) * h2
        y = jnp.dot(h, w3_ref[0],
                    preferred_element_type=jnp.float32) + b3_ref[0]
        out_ref[...] = y


def _run_gemm(tile_e, xs, W1, b1, W2, b2, W3, b3):
    grid_spec = pltpu.PrefetchScalarGridSpec(
        num_scalar_prefetch=1,
        grid=(NTILES,),
        in_specs=[
            pl.BlockSpec((BLK, DIM), lambda m, te: (m, 0)),
            pl.BlockSpec((1, DIM, FFN), lambda m, te: (te[m], 0, 0)),
            pl.BlockSpec((1, 1, FFN), lambda m, te: (te[m], 0, 0)),
            pl.BlockSpec((1, DIM, FFN), lambda m, te: (te[m], 0, 0)),
            pl.BlockSpec((1, 1, FFN), lambda m, te: (te[m], 0, 0)),
            pl.BlockSpec((1, FFN, DIM), lambda m, te: (te[m], 0, 0)),
            pl.BlockSpec((1, 1, DIM), lambda m, te: (te[m], 0, 0)),
        ],
        out_specs=pl.BlockSpec((BLK, DIM), lambda m, te: (m, 0)),
    )
    return pl.pallas_call(
        _gemm_kernel,
        grid_spec=grid_spec,
        out_shape=jax.ShapeDtypeStruct((P, DIM), jnp.float32),
        compiler_params=pltpu.CompilerParams(
            dimension_semantics=("arbitrary",),
        ),
    )(tile_e, xs, W1, b1[:, None, :], W2, b2[:, None, :], W3, b3[:, None, :])


def _make_dispatch():
    mesh = plsc.VectorSubcoreMesh(core_axis_name="c", subcore_axis_name="s")

    @functools.partial(
        pl.kernel, mesh=mesh,
        out_type=jax.ShapeDtypeStruct((P, DIM), jnp.float32),
        scratch_types=[
            pltpu.VMEM((TW,), jnp.int32),
            pltpu.VMEM((TW,), jnp.int32),
            pltpu.VMEM((TW, DIM), jnp.float32),
            pltpu.SemaphoreType.DMA,
        ],
    )
    def dispatch(d0_hbm, d1_hbm, x_hbm, xs_hbm, d0_v, d1_v, rows_v, sem):
        wid = lax.axis_index("s") * NC + lax.axis_index("c")
        base = wid * TW
        pltpu.sync_copy(d0_hbm.at[pl.ds(base, TW)], d0_v)
        pltpu.sync_copy(d1_hbm.at[pl.ds(base, TW)], d1_v)
        pltpu.sync_copy(x_hbm.at[pl.ds(base, TW)], rows_v)
        c0 = pltpu.async_copy(rows_v, xs_hbm.at[d0_v], sem)
        c1 = pltpu.async_copy(rows_v, xs_hbm.at[d1_v], sem)
        c0.wait()
        c1.wait()

    return dispatch


def _make_combine():
    mesh = plsc.VectorSubcoreMesh(core_axis_name="c", subcore_axis_name="s")

    @functools.partial(
        pl.kernel, mesh=mesh,
        out_type=jax.ShapeDtypeStruct((T, DIM), jnp.float32),
        scratch_types=[
            pltpu.VMEM((TW,), jnp.int32),
            pltpu.VMEM((TW,), jnp.int32),
            pltpu.VMEM((TW, L), jnp.float32),
            pltpu.VMEM((TW, L), jnp.float32),
            pltpu.VMEM((TW, DIM), jnp.float32),
            pltpu.VMEM((TW, DIM), jnp.float32),
            pltpu.SemaphoreType.DMA,
        ],
    )
    def combine(d0_hbm, d1_hbm, w0_hbm, w1_hbm, y_hbm, out_hbm,
                d0_v, d1_v, w0_v, w1_v, a_v, b_v, sem):
        wid = lax.axis_index("s") * NC + lax.axis_index("c")
        base = wid * TW
        pltpu.sync_copy(d0_hbm.at[pl.ds(base, TW)], d0_v)
        pltpu.sync_copy(d1_hbm.at[pl.ds(base, TW)], d1_v)
        pltpu.sync_copy(w0_hbm.at[pl.ds(base, TW)], w0_v)
        pltpu.sync_copy(w1_hbm.at[pl.ds(base, TW)], w1_v)
        hw = TW // 2
        cp = [None] * 4
        for s in range(2):
            ts = pl.ds(s * hw, hw)
            cp[2 * s] = pltpu.async_copy(
                y_hbm.at[d0_v.at[ts]], a_v.at[ts, :], sem)
            cp[2 * s + 1] = pltpu.async_copy(
                y_hbm.at[d1_v.at[ts]], b_v.at[ts, :], sem)

        def body(r, _):
            w0 = w0_v[r, :]
            w1 = w1_v[r, :]
            for c in range(DIM // L):
                sl = pl.ds(c * L, L)
                a_v[r, sl] = a_v[r, sl] * w0 + b_v[r, sl] * w1
            return 0

        cp[0].wait()
        cp[1].wait()
        lax.fori_loop(0, hw, body, 0)
        cp[2].wait()
        cp[3].wait()
        lax.fori_loop(hw, TW, body, 0)
        pltpu.sync_copy(a_v, out_hbm.at[pl.ds(base, TW)])

    return combine


_SC_KERNELS = {}


def _sc_kernels():
    if "dispatch" not in _SC_KERNELS:
        _SC_KERNELS["dispatch"] = _make_dispatch()
        _SC_KERNELS["combine"] = _make_combine()
    return _SC_KERNELS["dispatch"], _SC_KERNELS["combine"]


def kernel(x, Wr, W1, b1, W2, b2, W3, b3):
    dispatch, combine = _sc_kernels()
    xf = x.reshape(T, DIM)

    d0, d1, w0b, w1b, tile_e = _run_meta(xf, Wr)
    d0 = d0.reshape(T)
    d1 = d1.reshape(T)

    xs = dispatch(d0, d1, xf)

    y = _run_gemm(tile_e.reshape(NTILES + 1), xs, W1, b1, W2, b2, W3, b3)

    out = combine(d0, d1, w0b, w1b, y)

    return out.reshape(B, S, DIM)
